# baseline (speedup 1.0000x reference)
import functools
import math

import jax
import jax.numpy as jnp
from jax import lax
from jax.experimental import pallas as pl
from jax.experimental.pallas import tpu as pltpu
from jax.experimental.pallas import tpu_sc as plsc

S = 2048
D = 768
F = 3072
E = 8
K = 2
A = S * K
T = 512
NT = A // T + E
R = NT * T

NC = 2
NS = 16
NW = NC * NS
CH = A // NW
CT = S // NW
LANES = 16
WLANE = 128


def _gate_body(x_ref, wg_ref, bg_ref, pos_ref, pw_ref, te_ref, na_ref,
               aux_ref, eix_ref, nxe_ref, hnx_ref):
    x = x_ref[...]
    logits = jnp.dot(x, wg_ref[...], preferred_element_type=jnp.float32)
    logits = logits + bg_ref[...]
    iota = lax.broadcasted_iota(jnp.int32, (S, E), 1)
    m1 = jnp.max(logits, axis=1, keepdims=True)
    i1 = jnp.min(jnp.where(logits == m1, iota, E), axis=1, keepdims=True)
    rest = jnp.where(iota == i1, -jnp.inf, logits)
    m2 = jnp.max(rest, axis=1, keepdims=True)
    i2 = jnp.min(jnp.where(rest == m2, iota, E), axis=1, keepdims=True)
    p1 = 1.0 / (1.0 + jnp.exp(m2 - m1))
    z = jnp.exp(logits - m1)
    gp = z / jnp.sum(z, axis=1, keepdims=True)
    usage = jnp.mean(gp, axis=0)
    aux_ref[...] = (E * jnp.sum(usage * usage)).reshape(1, 1)

    i1t = jnp.transpose(i1)
    i2t = jnp.transpose(i2)
    eio = lax.broadcasted_iota(jnp.int32, (E, S), 0)
    a_t = (eio == i1t).astype(jnp.int32)
    b_t = (eio == i2t).astype(jnp.int32)
    s_t = a_t + b_t
    c = s_t
    sh = 1
    while sh < S:
        z2 = jnp.zeros((E, sh), jnp.int32)
        c = c + jnp.concatenate([z2, c[:, : S - sh]], axis=1)
        sh *= 2
    exc = c - s_t
    counts = c[:, S - 1 :]
    padded = ((counts + (T - 1)) // T) * T
    cp = padded
    sh = 1
    while sh < E:
        z2 = jnp.zeros((sh, 1), jnp.int32)
        cp = cp + jnp.concatenate([z2, cp[: E - sh, :]], axis=0)
        sh *= 2
    offs = cp - padded
    pos1 = jnp.sum((exc + offs) * a_t, axis=0, keepdims=True)
    pos2 = jnp.sum((exc + a_t + offs) * b_t, axis=0, keepdims=True)
    pos_ref[...] = jnp.concatenate([pos1, pos2], axis=0)
    na_ref[...] = cp[E - 1 :, :] // T
    tio = lax.broadcasted_iota(jnp.int32, (E, NT), 1) * T
    te = jnp.sum((tio >= cp).astype(jnp.int32), axis=0, keepdims=True)
    te_row = jnp.minimum(te, E - 1)
    te_ref[...] = te_row
    tprev = jnp.concatenate([te_row[:, :1], te_row[:, : NT - 1]], axis=1)
    lio = lax.broadcasted_iota(jnp.int32, (1, NT), 1)
    bnd = jnp.logical_or(lio == 0, te_row != tprev).astype(jnp.int32)
    eix = bnd
    sh = 1
    while sh < NT:
        z3 = jnp.zeros((1, sh), jnp.int32)
        eix = eix + jnp.concatenate([z3, eix[:, : NT - sh]], axis=1)
        sh *= 2
    eix = eix - 1
    na_b = cp[E - 1 :, :] // T
    nrun = jnp.sum(eix * (lio == na_b - 1), axis=1, keepdims=True) + 1
    ke = lax.broadcasted_iota(jnp.int32, (NT, NT), 0)
    eix_b = jnp.broadcast_to(eix, (NT, NT))
    run_exp = jnp.sum(jnp.broadcast_to(te_row, (NT, NT)) *
                      (eix_b == ke) * jnp.broadcast_to(bnd, (NT, NT)),
                      axis=1, keepdims=True)
    nxe = jnp.sum(run_exp * (ke == eix_b + 1), axis=0, keepdims=True)
    eix_ref[...] = eix
    nxe_ref[...] = nxe
    hnx_ref[...] = (eix + 1 < nrun).astype(jnp.int32)
    pw = jnp.concatenate([jnp.transpose(p1), 1.0 - jnp.transpose(p1)],
                         axis=0)
    pw_ref[...] = jnp.broadcast_to(pw[:, :, None], (K, S, WLANE))


def _gate(x2d, Wg, bg):
    return pl.pallas_call(
        _gate_body,
        out_shape=(
            jax.ShapeDtypeStruct((K, S), jnp.int32),
            jax.ShapeDtypeStruct((K, S, WLANE), jnp.float32),
            jax.ShapeDtypeStruct((1, NT), jnp.int32),
            jax.ShapeDtypeStruct((1, 1), jnp.int32),
            jax.ShapeDtypeStruct((1, 1), jnp.float32),
            jax.ShapeDtypeStruct((1, NT), jnp.int32),
            jax.ShapeDtypeStruct((1, NT), jnp.int32),
            jax.ShapeDtypeStruct((1, NT), jnp.int32),
        ),
    )(x2d, Wg, bg.reshape(1, E))


def _dispatch_body(x_hbm, tok_hbm, pos_hbm, pwb_hbm, xs_hbm, roww_hbm,
                   tok_v, pos_v, rows_v, w_v, sem_g, sem_s, sem_w):
    wid = lax.axis_index("s") * NC + lax.axis_index("c")
    base = wid * CH
    pltpu.sync_copy(tok_hbm.at[pl.ds(base, CH)], tok_v)
    pltpu.sync_copy(pos_hbm.at[pl.ds(base, CH)], pos_v)
    pltpu.sync_copy(pwb_hbm.at[pl.ds(base, CH)], w_v)
    gather = pltpu.async_copy(x_hbm.at[tok_v], rows_v, sem_g)
    wsc = pltpu.async_copy(w_v, roww_hbm.at[pos_v], sem_w)
    gather.wait()
    pltpu.async_copy(rows_v, xs_hbm.at[pos_v], sem_s).wait()
    wsc.wait()


def _dispatch(x2d, tok, pos, pwb):
    mesh = plsc.VectorSubcoreMesh(core_axis_name="c", subcore_axis_name="s")
    return pl.kernel(
        _dispatch_body,
        out_type=(
            jax.ShapeDtypeStruct((R, D), jnp.float32),
            jax.ShapeDtypeStruct((R, WLANE), jnp.float32),
        ),
        mesh=mesh,
        scratch_types=[
            pltpu.VMEM((CH,), jnp.int32),
            pltpu.VMEM((CH,), jnp.int32),
            pltpu.VMEM((CH, D), jnp.float32),
            pltpu.VMEM((CH, WLANE), jnp.float32),
            pltpu.SemaphoreType.DMA,
            pltpu.SemaphoreType.DMA,
            pltpu.SemaphoreType.DMA,
        ],
    )(x2d, tok, pos, pwb)


def _ffn_body(te_ref, na_ref, eix_ref, nxe_ref, hnx_ref, xs_ref, w1_any,
              b1_ref, w2_any, b2_ref, roww_ref, y_ref, w1r, w2r, sem1, sem2):
    i = pl.program_id(0)
    active = i < na_ref[0, 0]
    e_cur = te_ref[0, i]
    slot = lax.rem(eix_ref[0, i], 2)
    first = jnp.logical_or(i == 0,
                           e_cur != te_ref[0, jnp.maximum(i - 1, 0)])

    @pl.when(jnp.logical_and(active, first))
    def _():
        @pl.when(i == 0)
        def _():
            pltpu.make_async_copy(w1_any.at[e_cur], w1r.at[slot], sem1).start()
            pltpu.make_async_copy(w2_any.at[e_cur], w2r.at[slot], sem2).start()
        pltpu.make_async_copy(w1_any.at[e_cur], w1r.at[slot], sem1).wait()
        pltpu.make_async_copy(w2_any.at[e_cur], w2r.at[slot], sem2).wait()

        @pl.when(hnx_ref[0, i] == 1)
        def _():
            en = nxe_ref[0, i]
            pltpu.make_async_copy(w1_any.at[en], w1r.at[1 - slot], sem1).start()
            pltpu.make_async_copy(w2_any.at[en], w2r.at[1 - slot], sem2).start()

    @pl.when(active)
    def _():
        x = xs_ref[...]
        y = jnp.broadcast_to(b2_ref[0], (T, D))
        fb = F // 4
        for fc in range(4):
            sl = pl.ds(fc * fb, fb)
            h = jnp.dot(x, w1r[slot, :, sl],
                        preferred_element_type=jnp.float32,
                        precision=lax.Precision.DEFAULT)
            h = h + b1_ref[0, :, sl]
            h = 0.5 * h * (1.0 + lax.erf(h * (1.0 / math.sqrt(2.0))))
            y = y + jnp.dot(h, w2r[slot, sl, :],
                            preferred_element_type=jnp.float32,
                            precision=lax.Precision.DEFAULT)
        y_ref[...] = y * roww_ref[...][:, 0:1]


def _ffn(xs, W1, b1, W2, b2, roww, tile_expert, n_active, eix, nxe, hnx):
    grid_spec = pltpu.PrefetchScalarGridSpec(
        num_scalar_prefetch=5,
        grid=(NT,),
        in_specs=[
            pl.BlockSpec((T, D), lambda i, te, na, ei, nx, hn: (i, 0)),
            pl.BlockSpec(memory_space=pl.ANY),
            pl.BlockSpec((1, 1, F),
                         lambda i, te, na, ei, nx, hn: (te[0, i], 0, 0)),
            pl.BlockSpec(memory_space=pl.ANY),
            pl.BlockSpec((1, 1, D),
                         lambda i, te, na, ei, nx, hn: (te[0, i], 0, 0)),
            pl.BlockSpec((T, WLANE), lambda i, te, na, ei, nx, hn: (i, 0)),
        ],
        out_specs=pl.BlockSpec((T, D), lambda i, te, na, ei, nx, hn: (i, 0)),
        scratch_shapes=[
            pltpu.VMEM((2, D, F), jnp.float32),
            pltpu.VMEM((2, F, D), jnp.float32),
            pltpu.SemaphoreType.DMA,
            pltpu.SemaphoreType.DMA,
        ],
    )
    return pl.pallas_call(
        _ffn_body,
        grid_spec=grid_spec,
        out_shape=jax.ShapeDtypeStruct((R, D), jnp.float32),
    )(tile_expert, n_active, eix, nxe, hnx, xs, W1, b1.reshape(E, 1, F), W2,
      b2.reshape(E, 1, D), roww)


def _combine_body(y_hbm, pe_hbm, po_hbm, out_hbm, idxe_v, idxo_v, a_v, b_v,
                  sem_g, sem_a):
    wid = lax.axis_index("s") * NC + lax.axis_index("c")
    half_t = CT // 2
    for half in range(2):
        base = wid * CT + half * half_t
        pltpu.sync_copy(pe_hbm.at[pl.ds(base, half_t)], idxe_v)
        pltpu.sync_copy(po_hbm.at[pl.ds(base, half_t)], idxo_v)
        cp_a = pltpu.async_copy(y_hbm.at[idxe_v], a_v, sem_g)
        cp_b = pltpu.async_copy(y_hbm.at[idxo_v], b_v, sem_a)
        cp_a.wait()
        cp_b.wait()

        def body(j, _):
            for c in range(D // LANES):
                sl = pl.ds(c * LANES, LANES)
                a_v[j, sl] = a_v[j, sl] + b_v[j, sl]
            return 0

        lax.fori_loop(0, half_t, body, 0)
        pltpu.sync_copy(a_v, out_hbm.at[pl.ds(base, half_t)])


def _combine(y, pos_even, pos_odd):
    mesh = plsc.VectorSubcoreMesh(core_axis_name="c", subcore_axis_name="s")
    return pl.kernel(
        _combine_body,
        out_type=jax.ShapeDtypeStruct((S, D), jnp.float32),
        mesh=mesh,
        scratch_types=[
            pltpu.VMEM((CT // 2,), jnp.int32),
            pltpu.VMEM((CT // 2,), jnp.int32),
            pltpu.VMEM((CT // 2, D), jnp.float32),
            pltpu.VMEM((CT // 2, D), jnp.float32),
            pltpu.SemaphoreType.DMA,
            pltpu.SemaphoreType.DMA,
        ],
    )(y, pos_even, pos_odd)


def kernel(x, W1, b1, W2, b2, Wg, bg):
    x2d = x.reshape(S, D)
    pos_sm, pwb3, tile_expert, n_active, aux, eix, nxe, hnx = _gate(
        x2d, Wg, bg)
    tok = jnp.broadcast_to(jnp.arange(S, dtype=jnp.int32)[None, :],
                           (K, S)).reshape(A)
    xs, roww = _dispatch(x2d, tok, pos_sm.reshape(A), pwb3.reshape(A, WLANE))
    y = _ffn(xs, W1, b1, W2, b2, roww, tile_expert, n_active, eix, nxe, hnx)
    out2d = _combine(y, pos_sm[0].reshape(S), pos_sm[1].reshape(S))
    return out2d.reshape(1, S, D), aux.reshape(())


# speedup vs baseline: 1.0059x; 1.0059x over previous
import math

import jax
import jax.numpy as jnp
from jax import lax
from jax.experimental import pallas as pl
from jax.experimental.pallas import tpu as pltpu
from jax.experimental.pallas import tpu_sc as plsc

S = 2048
D = 768
F = 3072
E = 8
K = 2
A = S * K
T = 512
NT = A // T + E
R = NT * T

NC = 2
NS = 16
NW = NC * NS
CH = A // NW
CT = S // NW
LANES = 16
WLANE = 128


def _gate_body(x_ref, wg_ref, bg_ref, pos_ref, pw_ref, te_ref, na_ref,
               aux_ref, eix_ref, nxe_ref, hnx_ref):
    x = x_ref[...]
    logits = jnp.dot(x, wg_ref[...], preferred_element_type=jnp.float32)
    logits = logits + bg_ref[...]
    iota = lax.broadcasted_iota(jnp.int32, (S, E), 1)
    m1 = jnp.max(logits, axis=1, keepdims=True)
    i1 = jnp.min(jnp.where(logits == m1, iota, E), axis=1, keepdims=True)
    rest = jnp.where(iota == i1, -jnp.inf, logits)
    m2 = jnp.max(rest, axis=1, keepdims=True)
    i2 = jnp.min(jnp.where(rest == m2, iota, E), axis=1, keepdims=True)
    p1 = 1.0 / (1.0 + jnp.exp(m2 - m1))
    z = jnp.exp(logits - m1)
    gp = z / jnp.sum(z, axis=1, keepdims=True)
    usage = jnp.mean(gp, axis=0)
    aux_ref[...] = (E * jnp.sum(usage * usage)).reshape(1, 1)

    i1t = jnp.transpose(i1)
    i2t = jnp.transpose(i2)
    eio = lax.broadcasted_iota(jnp.int32, (E, S), 0)
    a_t = (eio == i1t).astype(jnp.int32)
    b_t = (eio == i2t).astype(jnp.int32)
    s_t = a_t + b_t
    c = s_t
    sh = 1
    while sh < S:
        z2 = jnp.zeros((E, sh), jnp.int32)
        c = c + jnp.concatenate([z2, c[:, : S - sh]], axis=1)
        sh *= 2
    exc = c - s_t
    counts = c[:, S - 1 :]
    padded = ((counts + (T - 1)) // T) * T
    cp = padded
    sh = 1
    while sh < E:
        z2 = jnp.zeros((sh, 1), jnp.int32)
        cp = cp + jnp.concatenate([z2, cp[: E - sh, :]], axis=0)
        sh *= 2
    offs = cp - padded
    pos1 = jnp.sum((exc + offs) * a_t, axis=0, keepdims=True)
    pos2 = jnp.sum((exc + a_t + offs) * b_t, axis=0, keepdims=True)
    pos_ref[...] = jnp.concatenate([pos1, pos2], axis=0)
    na_ref[...] = cp[E - 1 :, :] // T
    tio = lax.broadcasted_iota(jnp.int32, (E, NT), 1) * T
    te = jnp.sum((tio >= cp).astype(jnp.int32), axis=0, keepdims=True)
    te_row = jnp.minimum(te, E - 1)
    te_ref[...] = te_row
    tprev = jnp.concatenate([te_row[:, :1], te_row[:, : NT - 1]], axis=1)
    lio = lax.broadcasted_iota(jnp.int32, (1, NT), 1)
    bnd = jnp.logical_or(lio == 0, te_row != tprev).astype(jnp.int32)
    eix = bnd
    sh = 1
    while sh < NT:
        z3 = jnp.zeros((1, sh), jnp.int32)
        eix = eix + jnp.concatenate([z3, eix[:, : NT - sh]], axis=1)
        sh *= 2
    eix = eix - 1
    na_b = cp[E - 1 :, :] // T
    nrun = jnp.sum(eix * (lio == na_b - 1), axis=1, keepdims=True) + 1
    ke = lax.broadcasted_iota(jnp.int32, (NT, NT), 0)
    eix_b = jnp.broadcast_to(eix, (NT, NT))
    run_exp = jnp.sum(jnp.broadcast_to(te_row, (NT, NT)) *
                      (eix_b == ke) * jnp.broadcast_to(bnd, (NT, NT)),
                      axis=1, keepdims=True)
    nxe = jnp.sum(run_exp * (ke == eix_b + 1), axis=0, keepdims=True)
    eix_ref[...] = eix
    nxe_ref[...] = nxe
    hnx_ref[...] = (eix + 1 < nrun).astype(jnp.int32)
    pw = jnp.concatenate([jnp.transpose(p1), 1.0 - jnp.transpose(p1)],
                         axis=0)
    pw_ref[...] = jnp.broadcast_to(pw[:, :, None], (K, S, WLANE))


def _gate(x2d, Wg, bg):
    return pl.pallas_call(
        _gate_body,
        out_shape=(
            jax.ShapeDtypeStruct((K, S), jnp.int32),
            jax.ShapeDtypeStruct((K, S, WLANE), jnp.float32),
            jax.ShapeDtypeStruct((1, NT), jnp.int32),
            jax.ShapeDtypeStruct((1, 1), jnp.int32),
            jax.ShapeDtypeStruct((1, 1), jnp.float32),
            jax.ShapeDtypeStruct((1, NT), jnp.int32),
            jax.ShapeDtypeStruct((1, NT), jnp.int32),
            jax.ShapeDtypeStruct((1, NT), jnp.int32),
        ),
    )(x2d, Wg, bg.reshape(1, E))


def _dispatch_body(x_hbm, tok_hbm, pos_hbm, pwb_hbm, xs_hbm, roww_hbm,
                   tok_v, pos_v, rows_v, w_v, sem_g, sem_s, sem_w):
    wid = lax.axis_index("s") * NC + lax.axis_index("c")
    base = wid * CH
    pltpu.sync_copy(tok_hbm.at[pl.ds(base, CH)], tok_v)
    pltpu.sync_copy(pos_hbm.at[pl.ds(base, CH)], pos_v)
    pltpu.sync_copy(pwb_hbm.at[pl.ds(base, CH)], w_v)
    gather = pltpu.async_copy(x_hbm.at[tok_v], rows_v, sem_g)
    wsc = pltpu.async_copy(w_v, roww_hbm.at[pos_v], sem_w)
    gather.wait()
    pltpu.async_copy(rows_v, xs_hbm.at[pos_v], sem_s).wait()
    wsc.wait()


def _dispatch(x2d, tok, pos, pwb):
    mesh = plsc.VectorSubcoreMesh(core_axis_name="c", subcore_axis_name="s")
    return pl.kernel(
        _dispatch_body,
        out_type=(
            jax.ShapeDtypeStruct((R, D), jnp.float32),
            jax.ShapeDtypeStruct((R, WLANE), jnp.float32),
        ),
        mesh=mesh,
        scratch_types=[
            pltpu.VMEM((CH,), jnp.int32),
            pltpu.VMEM((CH,), jnp.int32),
            pltpu.VMEM((CH, D), jnp.float32),
            pltpu.VMEM((CH, WLANE), jnp.float32),
            pltpu.SemaphoreType.DMA,
            pltpu.SemaphoreType.DMA,
            pltpu.SemaphoreType.DMA,
        ],
    )(x2d, tok, pos, pwb)


def _ffn_body(te_ref, na_ref, eix_ref, nxe_ref, hnx_ref, xs_ref, w1_any,
              b1_ref, w2_any, b2_ref, roww_ref, y_ref, w1r, w2r, sem1, sem2):
    i = pl.program_id(0)
    active = i < na_ref[0, 0]
    e_cur = te_ref[0, i]
    slot = lax.rem(eix_ref[0, i], 2)
    first = jnp.logical_or(i == 0,
                           e_cur != te_ref[0, jnp.maximum(i - 1, 0)])

    @pl.when(jnp.logical_and(active, first))
    def _():
        @pl.when(i == 0)
        def _():
            pltpu.make_async_copy(w1_any.at[e_cur], w1r.at[slot], sem1).start()
            pltpu.make_async_copy(w2_any.at[e_cur], w2r.at[slot], sem2).start()
        pltpu.make_async_copy(w1_any.at[e_cur], w1r.at[slot], sem1).wait()
        pltpu.make_async_copy(w2_any.at[e_cur], w2r.at[slot], sem2).wait()

        @pl.when(hnx_ref[0, i] == 1)
        def _():
            en = nxe_ref[0, i]
            pltpu.make_async_copy(w1_any.at[en], w1r.at[1 - slot], sem1).start()
            pltpu.make_async_copy(w2_any.at[en], w2r.at[1 - slot], sem2).start()

    @pl.when(active)
    def _():
        x = xs_ref[...]
        h = jnp.dot(x, w1r[slot], preferred_element_type=jnp.float32,
                    precision=lax.Precision.DEFAULT)
        h = h + b1_ref[0]
        h = 0.5 * h * (1.0 + lax.erf(h * (1.0 / math.sqrt(2.0))))
        y = jnp.dot(h, w2r[slot], preferred_element_type=jnp.float32,
                    precision=lax.Precision.DEFAULT)
        y = y + b2_ref[0]
        y_ref[...] = y * roww_ref[...][:, 0:1]


def _ffn(xs, W1, b1, W2, b2, roww, tile_expert, n_active, eix, nxe, hnx):
    grid_spec = pltpu.PrefetchScalarGridSpec(
        num_scalar_prefetch=5,
        grid=(NT,),
        in_specs=[
            pl.BlockSpec((T, D), lambda i, te, na, ei, nx, hn: (i, 0)),
            pl.BlockSpec(memory_space=pl.ANY),
            pl.BlockSpec((1, 1, F),
                         lambda i, te, na, ei, nx, hn: (te[0, i], 0, 0)),
            pl.BlockSpec(memory_space=pl.ANY),
            pl.BlockSpec((1, 1, D),
                         lambda i, te, na, ei, nx, hn: (te[0, i], 0, 0)),
            pl.BlockSpec((T, WLANE), lambda i, te, na, ei, nx, hn: (i, 0)),
        ],
        out_specs=pl.BlockSpec((T, D), lambda i, te, na, ei, nx, hn: (i, 0)),
        scratch_shapes=[
            pltpu.VMEM((2, D, F), jnp.float32),
            pltpu.VMEM((2, F, D), jnp.float32),
            pltpu.SemaphoreType.DMA,
            pltpu.SemaphoreType.DMA,
        ],
    )
    return pl.pallas_call(
        _ffn_body,
        grid_spec=grid_spec,
        out_shape=jax.ShapeDtypeStruct((R, D), jnp.float32),
    )(tile_expert, n_active, eix, nxe, hnx, xs, W1, b1.reshape(E, 1, F), W2,
      b2.reshape(E, 1, D), roww)


def _combine_body(y_hbm, pe_hbm, po_hbm, out_hbm, idxe_v, idxo_v, a_v, b_v,
                  sem_g, sem_a):
    wid = lax.axis_index("s") * NC + lax.axis_index("c")
    half_t = CT // 2
    for half in range(2):
        base = wid * CT + half * half_t
        pltpu.sync_copy(pe_hbm.at[pl.ds(base, half_t)], idxe_v)
        pltpu.sync_copy(po_hbm.at[pl.ds(base, half_t)], idxo_v)
        cp_a = pltpu.async_copy(y_hbm.at[idxe_v], a_v, sem_g)
        cp_b = pltpu.async_copy(y_hbm.at[idxo_v], b_v, sem_a)
        cp_a.wait()
        cp_b.wait()

        def body(j, _):
            for c in range(D // LANES):
                sl = pl.ds(c * LANES, LANES)
                a_v[j, sl] = a_v[j, sl] + b_v[j, sl]
            return 0

        lax.fori_loop(0, half_t, body, 0)
        pltpu.sync_copy(a_v, out_hbm.at[pl.ds(base, half_t)])


def _combine(y, pos_even, pos_odd):
    mesh = plsc.VectorSubcoreMesh(core_axis_name="c", subcore_axis_name="s")
    return pl.kernel(
        _combine_body,
        out_type=jax.ShapeDtypeStruct((S, D), jnp.float32),
        mesh=mesh,
        scratch_types=[
            pltpu.VMEM((CT // 2,), jnp.int32),
            pltpu.VMEM((CT // 2,), jnp.int32),
            pltpu.VMEM((CT // 2, D), jnp.float32),
            pltpu.VMEM((CT // 2, D), jnp.float32),
            pltpu.SemaphoreType.DMA,
            pltpu.SemaphoreType.DMA,
        ],
    )(y, pos_even, pos_odd)


def kernel(x, W1, b1, W2, b2, Wg, bg):
    x2d = x.reshape(S, D)
    pos_sm, pwb3, tile_expert, n_active, aux, eix, nxe, hnx = _gate(
        x2d, Wg, bg)
    tok = jnp.broadcast_to(jnp.arange(S, dtype=jnp.int32)[None, :],
                           (K, S)).reshape(A)
    xs, roww = _dispatch(x2d, tok, pos_sm.reshape(A), pwb3.reshape(A, WLANE))
    y = _ffn(xs, W1, b1, W2, b2, roww, tile_expert, n_active, eix, nxe, hnx)
    out2d = _combine(y, pos_sm[0].reshape(S), pos_sm[1].reshape(S))
    return out2d.reshape(1, S, D), aux.reshape(())
